# Initial kernel scaffold; baseline (speedup 1.0000x reference)
import functools

import jax
import jax.numpy as jnp
from jax import lax
from jax.experimental import pallas as pl
from jax.experimental.pallas import tpu as pltpu
from jax.experimental.pallas import tpu_sc as plsc

N = 10000
E = 320000
D_IN = 128
D_H = 512
D_OUT = 40
D_OUT_PAD = 48

NC = 2
NS = 16
NW = NC * NS
EPW = E // NW
CH = 80
NCH = EPW // CH
RPS = N // NS
ZR = 125

_mesh = plsc.VectorSubcoreMesh(core_axis_name="c", subcore_axis_name="s")


def _zero_vmem(buf, rows, width):
    zv = jnp.zeros((16,), jnp.float32)

    @pl.loop(0, rows)
    def _(i):
        for j in range(width // 16):
            buf[i, pl.ds(j * 16, 16)] = zv


def _make_sc_deg():
    @functools.partial(
        pl.kernel,
        out_type=jax.ShapeDtypeStruct((NC, N, 16), jnp.float32),
        mesh=_mesh,
        scratch_types=[
            pltpu.VMEM((2, CH), jnp.int32),
            pltpu.VMEM((CH, 16), jnp.float32),
            pltpu.VMEM((ZR, 16), jnp.float32),
            pltpu.VMEM_SHARED((N, 16), jnp.float32),
        ],
    )
    def k(edges_hbm, out_hbm, ebuf, ones, zbuf, acc):
        cid = lax.axis_index("c")
        sid = lax.axis_index("s")

        _zero_vmem(zbuf, ZR, 16)
        ov = jnp.ones((16,), jnp.float32)

        @pl.loop(0, CH)
        def _(i):
            ones[i, pl.ds(0, 16)] = ov

        for t in range(RPS // ZR):
            pltpu.sync_copy(zbuf, acc.at[pl.ds(sid * RPS + t * ZR, ZR)])
        plsc.subcore_barrier()

        wid = sid * NC + cid
        base_w = wid * EPW

        @pl.loop(0, NCH)
        def _(j):
            pltpu.sync_copy(edges_hbm.at[:, pl.ds(base_w + j * CH, CH)], ebuf)
            pltpu.sync_copy(ones, acc.at[ebuf.at[1]], add=True)

        plsc.subcore_barrier()
        pltpu.sync_copy(
            acc.at[pl.ds(sid * RPS, RPS)],
            out_hbm.at[cid, pl.ds(sid * RPS, RPS)],
        )

    return k


def _make_sc_agg(D):
    @functools.partial(
        pl.kernel,
        out_type=jax.ShapeDtypeStruct((NC, N, D), jnp.float32),
        mesh=_mesh,
        scratch_types=[
            pltpu.VMEM((2, CH), jnp.int32),
            pltpu.VMEM((CH, D), jnp.float32),
            pltpu.VMEM((ZR, D), jnp.float32),
            pltpu.VMEM_SHARED((N, D), jnp.float32),
        ],
    )
    def k(xs_hbm, edges_hbm, out_hbm, ebuf, rows, zbuf, acc):
        cid = lax.axis_index("c")
        sid = lax.axis_index("s")

        _zero_vmem(zbuf, ZR, D)
        for t in range(RPS // ZR):
            pltpu.sync_copy(zbuf, acc.at[pl.ds(sid * RPS + t * ZR, ZR)])
        plsc.subcore_barrier()

        wid = sid * NC + cid
        base_w = wid * EPW

        @pl.loop(0, NCH)
        def _(j):
            pltpu.sync_copy(edges_hbm.at[:, pl.ds(base_w + j * CH, CH)], ebuf)
            pltpu.sync_copy(xs_hbm.at[ebuf.at[0]], rows)
            pltpu.sync_copy(rows, acc.at[ebuf.at[1]], add=True)

        plsc.subcore_barrier()
        pltpu.sync_copy(
            acc.at[pl.ds(sid * RPS, RPS)],
            out_hbm.at[cid, pl.ds(sid * RPS, RPS)],
        )

    return k


_sc_deg = _make_sc_deg()
_sc_agg_1 = _make_sc_agg(D_IN)
_sc_agg_2 = _make_sc_agg(D_OUT_PAD)

BM = 400
GM = N // BM


def _tc_a_body(d0_ref, d1_ref, x_ref, dinv_ref, xs1_ref):
    deg = d0_ref[:, 0:1] + d1_ref[:, 0:1] + 1.0
    dinv = lax.rsqrt(deg)
    dinv_ref[...] = dinv
    xs1_ref[...] = x_ref[...] * dinv


def _tc_b_body(p0_ref, p1_ref, xs1_ref, dinv_ref, w1_ref, b1_ref, w2_ref,
               h1_ref, xs2_ref):
    dinv = dinv_ref[...]
    a1 = (p0_ref[...] + p1_ref[...] + xs1_ref[...]) * dinv
    h1 = jnp.maximum(
        lax.dot_general(a1, w1_ref[...], (((1,), (0,)), ((), ())),
                        precision=lax.Precision.HIGHEST,
                        preferred_element_type=jnp.float32) + b1_ref[...],
        0.0)
    h1_ref[...] = h1
    xs2_ref[...] = lax.dot_general(h1, w2_ref[...], (((1,), (0,)), ((), ())),
                                   precision=lax.Precision.HIGHEST,
                                   preferred_element_type=jnp.float32) * dinv


def _tc_c_body(q0_ref, q1_ref, xs2_ref, dinv_ref, b2_ref, out_ref):
    s = (q0_ref[...] + q1_ref[...] + xs2_ref[...])[:, :D_OUT]
    out_ref[...] = s * dinv_ref[...] + b2_ref[...]


def _row_spec(d):
    return pl.BlockSpec((BM, d), lambda i: (i, 0))


def _full_spec(shape):
    return pl.BlockSpec(shape, lambda i: tuple(0 for _ in shape))


_tc_a = pl.pallas_call(
    _tc_a_body,
    grid=(GM,),
    in_specs=[_row_spec(16), _row_spec(16), _row_spec(D_IN)],
    out_specs=[_row_spec(1), _row_spec(D_IN)],
    out_shape=[
        jax.ShapeDtypeStruct((N, 1), jnp.float32),
        jax.ShapeDtypeStruct((N, D_IN), jnp.float32),
    ],
)

_tc_b = pl.pallas_call(
    _tc_b_body,
    grid=(GM,),
    in_specs=[
        _row_spec(D_IN), _row_spec(D_IN), _row_spec(D_IN), _row_spec(1),
        _full_spec((D_IN, D_H)), _full_spec((1, D_H)),
        _full_spec((D_H, D_OUT_PAD)),
    ],
    out_specs=[_row_spec(D_H), _row_spec(D_OUT_PAD)],
    out_shape=[
        jax.ShapeDtypeStruct((N, D_H), jnp.float32),
        jax.ShapeDtypeStruct((N, D_OUT_PAD), jnp.float32),
    ],
)

_tc_c = pl.pallas_call(
    _tc_c_body,
    grid=(GM,),
    in_specs=[
        _row_spec(D_OUT_PAD), _row_spec(D_OUT_PAD), _row_spec(D_OUT_PAD),
        _row_spec(1), _full_spec((1, D_OUT)),
    ],
    out_specs=_row_spec(D_OUT),
    out_shape=jax.ShapeDtypeStruct((N, D_OUT), jnp.float32),
)


@jax.jit
def kernel(x, edge_index, W1, b1, W2, b2):
    deg_parts = _sc_deg(edge_index)
    dinv, xs1 = _tc_a(deg_parts[0], deg_parts[1], x)
    agg1 = _sc_agg_1(xs1, edge_index)
    w2p = jnp.zeros((D_H, D_OUT_PAD), jnp.float32).at[:, :D_OUT].set(W2)
    h1, xs2 = _tc_b(agg1[0], agg1[1], xs1, dinv, W1,
                    b1.reshape(1, D_H), w2p)
    agg2 = _sc_agg_2(xs2, edge_index)
    out = _tc_c(agg2[0], agg2[1], xs2, dinv, b2.reshape(1, D_OUT))
    return (out, h1)


# speedup vs baseline: 14.9954x; 14.9954x over previous
import functools

import jax
import jax.numpy as jnp
from jax import lax
from jax.experimental import pallas as pl
from jax.experimental.pallas import tpu as pltpu
from jax.experimental.pallas import tpu_sc as plsc

N = 10000
E = 320000
D_IN = 128
D_H = 512
D_OUT = 40

NC = 2
NS = 16
NW = NC * NS
EPW = E // NW
CH = 80
NCH = EPW // CH
N_PAD = 10112
RPS = N_PAD // NS
ZR = 104

_mesh = plsc.VectorSubcoreMesh(core_axis_name="c", subcore_axis_name="s")


def _zero_vmem(buf, rows, width):
    zv = jnp.zeros((16,), jnp.float32)

    @pl.loop(0, rows)
    def _(i):
        for j in range(width // 16):
            buf[i, pl.ds(j * 16, 16)] = zv


def _make_sc_deg():
    @functools.partial(
        pl.kernel,
        out_type=jax.ShapeDtypeStruct((NC, N_PAD, 128), jnp.float32),
        mesh=_mesh,
        scratch_types=[
            pltpu.VMEM((CH,), jnp.int32),
            pltpu.VMEM((CH, 128), jnp.float32),
            pltpu.VMEM((ZR, 128), jnp.float32),
            pltpu.VMEM_SHARED((N_PAD, 128), jnp.float32),
        ],
    )
    def k(dst_hbm, out_hbm, dbuf, ones, zbuf, acc):
        cid = lax.axis_index("c")
        sid = lax.axis_index("s")

        _zero_vmem(zbuf, ZR, 128)
        ov = jnp.ones((16,), jnp.float32)

        @pl.loop(0, CH)
        def _(i):
            for j in range(8):
                ones[i, pl.ds(j * 16, 16)] = ov

        start = sid * RPS
        for t in range(RPS // ZR):
            pltpu.sync_copy(zbuf, acc.at[pl.ds(start + t * ZR, ZR)])
        pltpu.sync_copy(zbuf.at[pl.ds(0, RPS % ZR)],
                        acc.at[pl.ds(start + (RPS // ZR) * ZR, RPS % ZR)])
        plsc.subcore_barrier()

        wid = sid * NC + cid
        base_w = wid * EPW

        @pl.loop(0, NCH)
        def _(j):
            pltpu.sync_copy(dst_hbm.at[pl.ds(base_w + j * CH, CH)], dbuf)
            pltpu.sync_copy(ones, acc.at[dbuf], add=True)

        plsc.subcore_barrier()
        pltpu.sync_copy(acc.at[pl.ds(start, RPS)],
                        out_hbm.at[cid, pl.ds(start, RPS)])

    return k


def _make_sc_agg(D):
    @functools.partial(
        pl.kernel,
        out_type=jax.ShapeDtypeStruct((NC, N_PAD, D), jnp.float32),
        mesh=_mesh,
        scratch_types=[
            pltpu.VMEM((CH,), jnp.int32),
            pltpu.VMEM((CH,), jnp.int32),
            pltpu.VMEM((CH, D), jnp.float32),
            pltpu.VMEM((ZR, D), jnp.float32),
            pltpu.VMEM_SHARED((N_PAD, D), jnp.float32),
        ],
    )
    def k(xs_hbm, src_hbm, dst_hbm, out_hbm, sbuf, dbuf, rows, zbuf, acc):
        cid = lax.axis_index("c")
        sid = lax.axis_index("s")

        _zero_vmem(zbuf, ZR, D)
        start = sid * RPS
        for t in range(RPS // ZR):
            pltpu.sync_copy(zbuf, acc.at[pl.ds(start + t * ZR, ZR)])
        pltpu.sync_copy(zbuf.at[pl.ds(0, RPS % ZR)],
                        acc.at[pl.ds(start + (RPS // ZR) * ZR, RPS % ZR)])
        plsc.subcore_barrier()

        wid = sid * NC + cid
        base_w = wid * EPW

        @pl.loop(0, NCH)
        def _(j):
            pltpu.sync_copy(src_hbm.at[pl.ds(base_w + j * CH, CH)], sbuf)
            pltpu.sync_copy(dst_hbm.at[pl.ds(base_w + j * CH, CH)], dbuf)
            pltpu.sync_copy(xs_hbm.at[sbuf], rows)
            pltpu.sync_copy(rows, acc.at[dbuf], add=True)

        plsc.subcore_barrier()
        pltpu.sync_copy(acc.at[pl.ds(start, RPS)],
                        out_hbm.at[cid, pl.ds(start, RPS)])

    return k


D_OUT_PAD = 128

_sc_deg = _make_sc_deg()
_sc_agg = _make_sc_agg(D_IN)
_sc_agg2 = _sc_agg

BM = 400
GM = N // BM


def _tc_a_body(d0_ref, d1_ref, x_ref, dinv_ref, xs1_ref):
    deg = d0_ref[:, 0:1] + d1_ref[:, 0:1] + 1.0
    dinv = lax.rsqrt(deg)
    dinv_ref[...] = dinv
    xs1_ref[...] = x_ref[...] * dinv


def _tc_b_body(p0_ref, p1_ref, xs1_ref, dinv_ref, w1_ref, b1_ref, w2_ref,
               h1_ref, xs2_ref):
    dinv = dinv_ref[...]
    a1 = (p0_ref[...] + p1_ref[...] + xs1_ref[...]) * dinv
    h1 = jnp.maximum(
        lax.dot_general(a1, w1_ref[...], (((1,), (0,)), ((), ())),
                        precision=lax.Precision.HIGHEST,
                        preferred_element_type=jnp.float32) + b1_ref[...],
        0.0)
    h1_ref[...] = h1
    xs2_ref[...] = lax.dot_general(h1, w2_ref[...], (((1,), (0,)), ((), ())),
                                   precision=lax.Precision.HIGHEST,
                                   preferred_element_type=jnp.float32) * dinv


def _tc_c_body(q0_ref, q1_ref, xs2_ref, dinv_ref, b2_ref, out_ref):
    s = (q0_ref[...] + q1_ref[...] + xs2_ref[...])[:, :D_OUT]
    out_ref[...] = s * dinv_ref[...] + b2_ref[...]


def _row_spec(d):
    return pl.BlockSpec((BM, d), lambda i: (i, 0))


def _full_spec(shape):
    return pl.BlockSpec(shape, lambda i: tuple(0 for _ in shape))


_tc_a = pl.pallas_call(
    _tc_a_body,
    grid=(GM,),
    in_specs=[_row_spec(D_IN), _row_spec(D_IN), _row_spec(D_IN)],
    out_specs=[_row_spec(1), _row_spec(D_IN)],
    out_shape=[
        jax.ShapeDtypeStruct((N, 1), jnp.float32),
        jax.ShapeDtypeStruct((N, D_IN), jnp.float32),
    ],
)

_tc_b = pl.pallas_call(
    _tc_b_body,
    grid=(GM,),
    in_specs=[
        _row_spec(D_IN), _row_spec(D_IN), _row_spec(D_IN), _row_spec(1),
        _full_spec((D_IN, D_H)), _full_spec((1, D_H)),
        _full_spec((D_H, D_OUT_PAD)),
    ],
    out_specs=[_row_spec(D_H), _row_spec(D_OUT_PAD)],
    out_shape=[
        jax.ShapeDtypeStruct((N, D_H), jnp.float32),
        jax.ShapeDtypeStruct((N, D_OUT_PAD), jnp.float32),
    ],
)

_tc_c = pl.pallas_call(
    _tc_c_body,
    grid=(GM,),
    in_specs=[
        _row_spec(D_OUT_PAD), _row_spec(D_OUT_PAD), _row_spec(D_OUT_PAD),
        _row_spec(1), _full_spec((1, D_OUT)),
    ],
    out_specs=_row_spec(D_OUT),
    out_shape=jax.ShapeDtypeStruct((N, D_OUT), jnp.float32),
)


@jax.jit
def kernel(x, edge_index, W1, b1, W2, b2):
    src = edge_index[0]
    dst = edge_index[1]
    deg_parts = _sc_deg(dst)
    dinv, xs1 = _tc_a(deg_parts[0], deg_parts[1], x)
    agg1 = _sc_agg(xs1, src, dst)
    w2p = jnp.zeros((D_H, D_OUT_PAD), jnp.float32).at[:, :D_OUT].set(W2)
    h1, xs2 = _tc_b(agg1[0], agg1[1], xs1, dinv, W1,
                    b1.reshape(1, D_H), w2p)
    agg2 = _sc_agg2(xs2, src, dst)
    out = _tc_c(agg2[0], agg2[1], xs2, dinv, b2.reshape(1, D_OUT))
    return (out, h1)


# speedup vs baseline: 28.4647x; 1.8982x over previous
import dataclasses
import functools

import jax
import jax.numpy as jnp
from jax import lax
from jax.experimental import pallas as pl
from jax.experimental.pallas import tpu as pltpu
from jax.experimental.pallas import tpu_sc as plsc

N = 10000
E = 320000
D_IN = 128
D_H = 512
D_OUT = 40

NC = 2
NS = 16
NW = NC * NS
EPW = E // NW
CH = 80
NCH = EPW // CH
N_PAD = 10112
RPS = N_PAD // NS
ZR = 104

_mesh = plsc.VectorSubcoreMesh(core_axis_name="c", subcore_axis_name="s")

_sc_cp = pltpu.CompilerParams()
if "needs_layout_passes" in pltpu.CompilerParams.__dataclass_fields__:
    _sc_cp = dataclasses.replace(_sc_cp, needs_layout_passes=False)


def _zero_vmem(buf, rows, width):
    zv = jnp.zeros((16,), jnp.float32)

    @pl.loop(0, rows)
    def _(i):
        for j in range(width // 16):
            buf[i, pl.ds(j * 16, 16)] = zv


def _make_sc_deg():
    @functools.partial(
        pl.kernel,
        out_type=jax.ShapeDtypeStruct((NW * N,), jnp.float32),
        mesh=_mesh,
        compiler_params=_sc_cp,
        scratch_types=[
            pltpu.VMEM((EPW,), jnp.int32),
            pltpu.VMEM((N,), jnp.float32),
        ],
    )
    def k(dst_hbm, out_hbm, dbuf, hist):
        cid = lax.axis_index("c")
        sid = lax.axis_index("s")
        wid = sid * NC + cid
        zv = jnp.zeros((16,), jnp.float32)

        @pl.loop(0, N // 16)
        def _(i):
            hist[pl.ds(i * 16, 16)] = zv

        pltpu.sync_copy(dst_hbm.at[pl.ds(wid * EPW, EPW)], dbuf)
        ov = jnp.ones((16,), jnp.float32)

        @pl.loop(0, EPW // 16)
        def _(i):
            idx = dbuf[pl.ds(i * 16, 16)]
            plsc.addupdate_scatter(hist, [idx], ov)

        pltpu.sync_copy(hist, out_hbm.at[pl.ds(wid * N, N)])

    return k


NB = 4
NG = NCH // NB


def _make_sc_agg_real(D):
    @functools.partial(
        pl.kernel,
        out_type=jax.ShapeDtypeStruct((NC, N_PAD, D), jnp.float32),
        mesh=_mesh,
        scratch_types=[
            pltpu.VMEM((NB, CH), jnp.int32),
            pltpu.VMEM((NB, CH), jnp.int32),
            [pltpu.VMEM((CH, D), jnp.float32) for _ in range(NB)],
            pltpu.VMEM_SHARED((N_PAD, D), jnp.float32),
            pltpu.SemaphoreType.DMA,
            pltpu.SemaphoreType.DMA,
            pltpu.SemaphoreType.DMA,
        ],
    )
    def k(xs_hbm, src_hbm, dst_hbm, out_hbm, sbuf, dbuf, rows, acc,
          isem, gsem, ssem):
        cid = lax.axis_index("c")
        sid = lax.axis_index("s")

        _zero_vmem(rows[0], CH, D)
        start = sid * RPS
        zh = []
        for t in range(RPS // CH):
            zh.append(pltpu.async_copy(
                rows[0], acc.at[pl.ds(start + t * CH, CH)], ssem))
        zh.append(pltpu.async_copy(
            rows[0].at[pl.ds(0, RPS % CH)],
            acc.at[pl.ds(start + (RPS // CH) * CH, RPS % CH)], ssem))
        for h in zh:
            h.wait()
        plsc.subcore_barrier()

        wid = sid * NC + cid
        base_w = wid * EPW

        @pl.loop(0, NG)
        def _(g):
            base = base_w + g * (NB * CH)
            ih = []
            for b in range(NB):
                ih.append(pltpu.async_copy(
                    src_hbm.at[pl.ds(base + b * CH, CH)], sbuf.at[b], isem))
                ih.append(pltpu.async_copy(
                    dst_hbm.at[pl.ds(base + b * CH, CH)], dbuf.at[b], isem))
            for h in ih:
                h.wait()
            gh = [pltpu.async_copy(xs_hbm.at[sbuf.at[b]], rows[b], gsem)
                  for b in range(NB)]
            for h in gh:
                h.wait()
            sh = [pltpu.async_copy(rows[b], acc.at[dbuf.at[b]], ssem,
                                   add=True)
                  for b in range(NB)]
            for h in sh:
                h.wait()

        for r in range(NCH - NG * NB):
            base = base_w + (NG * NB + r) * CH
            pltpu.sync_copy(src_hbm.at[pl.ds(base, CH)], sbuf.at[0])
            pltpu.sync_copy(dst_hbm.at[pl.ds(base, CH)], dbuf.at[0])
            pltpu.sync_copy(xs_hbm.at[sbuf.at[0]], rows[0])
            pltpu.sync_copy(rows[0], acc.at[dbuf.at[0]], add=True)

        plsc.subcore_barrier()
        pltpu.sync_copy(acc.at[pl.ds(start, RPS)],
                        out_hbm.at[cid, pl.ds(start, RPS)])

    return k


D_OUT_PAD = 128

_sc_deg = _make_sc_deg()
_sc_agg = _make_sc_agg_real(D_IN)
_sc_agg2 = _sc_agg

BM = 400
GM = N // BM


def _tc_a_body(dp_ref, x_ref, dinv_ref, xs1_ref):
    ones_w = jnp.ones((NW, 1), jnp.float32)
    deg = lax.dot_general(dp_ref[...], ones_w, (((0,), (0,)), ((), ())),
                          precision=lax.Precision.HIGHEST,
                          preferred_element_type=jnp.float32) + 1.0
    dinv = lax.rsqrt(deg)
    dinv_ref[...] = dinv
    xs1_ref[...] = x_ref[...] * dinv


def _tc_b_body(p0_ref, p1_ref, xs1_ref, dinv_ref, w1_ref, b1_ref, w2_ref,
               h1_ref, xs2_ref):
    dinv = dinv_ref[...]
    a1 = (p0_ref[...] + p1_ref[...] + xs1_ref[...]) * dinv
    h1 = jnp.maximum(
        lax.dot_general(a1, w1_ref[...], (((1,), (0,)), ((), ())),
                        precision=lax.Precision.HIGHEST,
                        preferred_element_type=jnp.float32) + b1_ref[...],
        0.0)
    h1_ref[...] = h1
    xs2_ref[...] = lax.dot_general(h1, w2_ref[...], (((1,), (0,)), ((), ())),
                                   precision=lax.Precision.HIGHEST,
                                   preferred_element_type=jnp.float32) * dinv


def _tc_c_body(q0_ref, q1_ref, xs2_ref, dinv_ref, b2_ref, out_ref):
    s = (q0_ref[...] + q1_ref[...] + xs2_ref[...])[:, :D_OUT]
    out_ref[...] = s * dinv_ref[...] + b2_ref[...]


def _row_spec(d):
    return pl.BlockSpec((BM, d), lambda i: (i, 0))


def _full_spec(shape):
    return pl.BlockSpec(shape, lambda i: tuple(0 for _ in shape))


_tc_a = pl.pallas_call(
    _tc_a_body,
    out_shape=[
        jax.ShapeDtypeStruct((N, 1), jnp.float32),
        jax.ShapeDtypeStruct((N, D_IN), jnp.float32),
    ],
)

_tc_b = pl.pallas_call(
    _tc_b_body,
    grid=(GM,),
    in_specs=[
        _row_spec(D_IN), _row_spec(D_IN), _row_spec(D_IN), _row_spec(1),
        _full_spec((D_IN, D_H)), _full_spec((1, D_H)),
        _full_spec((D_H, D_OUT_PAD)),
    ],
    out_specs=[_row_spec(D_H), _row_spec(D_OUT_PAD)],
    out_shape=[
        jax.ShapeDtypeStruct((N, D_H), jnp.float32),
        jax.ShapeDtypeStruct((N, D_OUT_PAD), jnp.float32),
    ],
)

_tc_c = pl.pallas_call(
    _tc_c_body,
    grid=(GM,),
    in_specs=[
        _row_spec(D_OUT_PAD), _row_spec(D_OUT_PAD), _row_spec(D_OUT_PAD),
        _row_spec(1), _full_spec((1, D_OUT)),
    ],
    out_specs=_row_spec(D_OUT),
    out_shape=jax.ShapeDtypeStruct((N, D_OUT), jnp.float32),
)


@jax.jit
def kernel(x, edge_index, W1, b1, W2, b2):
    src = edge_index[0]
    dst = edge_index[1]
    deg_parts = _sc_deg(dst).reshape(NW, N)
    dinv, xs1 = _tc_a(deg_parts, x)
    agg1 = _sc_agg(xs1, src, dst)
    w2p = jnp.zeros((D_H, D_OUT_PAD), jnp.float32).at[:, :D_OUT].set(W2)
    h1, xs2 = _tc_b(agg1[0], agg1[1], xs1, dinv, W1,
                    b1.reshape(1, D_H), w2p)
    agg2 = _sc_agg2(xs2, src, dst)
    out = _tc_c(agg2[0], agg2[1], xs2, dinv, b2.reshape(1, D_OUT))
    return (out, h1)


# speedup vs baseline: 35.7764x; 1.2569x over previous
import dataclasses
import functools

import jax
import jax.numpy as jnp
from jax import lax
from jax.experimental import pallas as pl
from jax.experimental.pallas import tpu as pltpu
from jax.experimental.pallas import tpu_sc as plsc

N = 10000
E = 320000
D_IN = 128
D_H = 512
D_OUT = 40

NC = 2
NS = 16
NW = NC * NS
EPW = E // NW
CH = 80
NCH = EPW // CH
N_PAD = 10112
RPS = N_PAD // NS
ZR = 104

_mesh = plsc.VectorSubcoreMesh(core_axis_name="c", subcore_axis_name="s")

_sc_cp = pltpu.CompilerParams()
if "needs_layout_passes" in pltpu.CompilerParams.__dataclass_fields__:
    _sc_cp = dataclasses.replace(_sc_cp, needs_layout_passes=False)


def _zero_vmem(buf, rows, width):
    zv = jnp.zeros((16,), jnp.float32)

    @pl.loop(0, rows)
    def _(i):
        for j in range(width // 16):
            buf[i, pl.ds(j * 16, 16)] = zv


def _make_sc_deg():
    @functools.partial(
        pl.kernel,
        out_type=jax.ShapeDtypeStruct((NW * N,), jnp.float32),
        mesh=_mesh,
        compiler_params=_sc_cp,
        scratch_types=[
            pltpu.VMEM((EPW,), jnp.int32),
            pltpu.VMEM((N,), jnp.float32),
        ],
    )
    def k(dst_hbm, out_hbm, dbuf, hist):
        cid = lax.axis_index("c")
        sid = lax.axis_index("s")
        wid = sid * NC + cid
        zv = jnp.zeros((16,), jnp.float32)

        @pl.loop(0, N // 16)
        def _(i):
            hist[pl.ds(i * 16, 16)] = zv

        pltpu.sync_copy(dst_hbm.at[pl.ds(wid * EPW, EPW)], dbuf)
        ov = jnp.ones((16,), jnp.float32)

        @pl.loop(0, EPW // 16)
        def _(i):
            idx = dbuf[pl.ds(i * 16, 16)]
            plsc.addupdate_scatter(hist, [idx], ov)

        pltpu.sync_copy(hist, out_hbm.at[pl.ds(wid * N, N)])

    return k


NB = 4
NG = NCH // NB


def _make_sc_agg_real(D):
    @functools.partial(
        pl.kernel,
        out_type=jax.ShapeDtypeStruct((NC, N_PAD, D), jnp.float32),
        mesh=_mesh,
        scratch_types=[
            pltpu.VMEM((2, NB, CH), jnp.int32),
            pltpu.VMEM((3, NB, CH), jnp.int32),
            [pltpu.VMEM((CH, D), jnp.float32) for _ in range(NB)],
            pltpu.VMEM_SHARED((N_PAD, D), jnp.float32),
            pltpu.SemaphoreType.DMA,
            [pltpu.SemaphoreType.DMA for _ in range(NB)],
            [pltpu.SemaphoreType.DMA for _ in range(NB)],
        ],
    )
    def k(xs_hbm, src_hbm, dst_hbm, out_hbm, sidx, didx, rows, acc,
          isem, gsem, ssem):
        cid = lax.axis_index("c")
        sid = lax.axis_index("s")

        _zero_vmem(rows[0], CH, D)
        start = sid * RPS
        zh = []
        for t in range(RPS // CH):
            zh.append(pltpu.async_copy(
                rows[0], acc.at[pl.ds(start + t * CH, CH)], ssem[t % NB]))
        zh.append(pltpu.async_copy(
            rows[0].at[pl.ds(0, RPS % CH)],
            acc.at[pl.ds(start + (RPS // CH) * CH, RPS % CH)], ssem[0]))
        for h in zh:
            h.wait()
        plsc.subcore_barrier()

        wid = sid * NC + cid
        base_w = wid * EPW

        def idx_copies(g, s2, s3, issue):
            base = base_w + g * (NB * CH)
            hs = []
            for b in range(NB):
                pairs = [
                    (src_hbm.at[pl.ds(base + b * CH, CH)], sidx.at[s2, b]),
                    (dst_hbm.at[pl.ds(base + b * CH, CH)], didx.at[s3, b]),
                ]
                for sref, dref in pairs:
                    if issue:
                        hs.append(pltpu.async_copy(sref, dref, isem))
                    else:
                        pltpu.make_async_copy(sref, dref, isem).wait()
            return hs

        idx_copies(0, 0, 0, issue=True)

        @pl.loop(0, NG)
        def _(g):
            s2 = lax.rem(g, 2)
            s3 = lax.rem(g, 3)

            @pl.when(g < NG - 1)
            def _():
                idx_copies(g + 1, lax.rem(g + 1, 2), lax.rem(g + 1, 3),
                           issue=True)

            idx_copies(g, s2, s3, issue=False)

            ps3 = lax.rem(g + 2, 3)
            for b in range(NB):
                @pl.when(g > 0)
                def _():
                    pltpu.make_async_copy(
                        rows[b], acc.at[didx.at[ps3, b]], ssem[b]).wait()
                pltpu.async_copy(xs_hbm.at[sidx.at[s2, b]], rows[b], gsem[b])
            for b in range(NB):
                pltpu.make_async_copy(
                    xs_hbm.at[sidx.at[s2, b]], rows[b], gsem[b]).wait()
                pltpu.async_copy(rows[b], acc.at[didx.at[s3, b]], ssem[b],
                                 add=True)

        ls3 = (NG - 1) % 3
        for b in range(NB):
            pltpu.make_async_copy(
                rows[b], acc.at[didx.at[ls3, b]], ssem[b]).wait()

        for r in range(NCH - NG * NB):
            base = base_w + (NG * NB + r) * CH
            pltpu.sync_copy(src_hbm.at[pl.ds(base, CH)], sidx.at[0, 0])
            pltpu.sync_copy(dst_hbm.at[pl.ds(base, CH)], didx.at[0, 0])
            pltpu.sync_copy(xs_hbm.at[sidx.at[0, 0]], rows[0])
            pltpu.sync_copy(rows[0], acc.at[didx.at[0, 0]], add=True)

        plsc.subcore_barrier()
        pltpu.sync_copy(acc.at[pl.ds(start, RPS)],
                        out_hbm.at[cid, pl.ds(start, RPS)])

    return k


D_OUT_PAD = 128

_sc_deg = _make_sc_deg()
_sc_agg = _make_sc_agg_real(D_IN)
_sc_agg2 = _sc_agg

BM = 400
GM = N // BM


def _tc_a_body(dp_ref, x_ref, dinv_ref, xs1_ref):
    ones_w = jnp.ones((NW, 1), jnp.float32)
    deg = lax.dot_general(dp_ref[...], ones_w, (((0,), (0,)), ((), ())),
                          precision=lax.Precision.HIGHEST,
                          preferred_element_type=jnp.float32) + 1.0
    dinv = lax.rsqrt(deg)
    dinv_ref[...] = dinv
    xs1_ref[...] = x_ref[...] * dinv


def _tc_b_body(p0_ref, p1_ref, xs1_ref, dinv_ref, w1_ref, b1_ref, w2_ref,
               h1_ref, xs2_ref):
    dinv = dinv_ref[...]
    a1 = (p0_ref[...] + p1_ref[...] + xs1_ref[...]) * dinv
    h1 = jnp.maximum(
        lax.dot_general(a1, w1_ref[...], (((1,), (0,)), ((), ())),
                        precision=lax.Precision.HIGHEST,
                        preferred_element_type=jnp.float32) + b1_ref[...],
        0.0)
    h1_ref[...] = h1
    xs2_ref[...] = lax.dot_general(h1, w2_ref[...], (((1,), (0,)), ((), ())),
                                   precision=lax.Precision.HIGHEST,
                                   preferred_element_type=jnp.float32) * dinv


def _tc_c_body(q0_ref, q1_ref, xs2_ref, dinv_ref, b2_ref, out_ref):
    s = (q0_ref[...] + q1_ref[...] + xs2_ref[...])[:, :D_OUT]
    out_ref[...] = s * dinv_ref[...] + b2_ref[...]


def _row_spec(d):
    return pl.BlockSpec((BM, d), lambda i: (i, 0))


def _full_spec(shape):
    return pl.BlockSpec(shape, lambda i: tuple(0 for _ in shape))


_tc_a = pl.pallas_call(
    _tc_a_body,
    out_shape=[
        jax.ShapeDtypeStruct((N, 1), jnp.float32),
        jax.ShapeDtypeStruct((N, D_IN), jnp.float32),
    ],
)

_tc_b = pl.pallas_call(
    _tc_b_body,
    grid=(GM,),
    in_specs=[
        _row_spec(D_IN), _row_spec(D_IN), _row_spec(D_IN), _row_spec(1),
        _full_spec((D_IN, D_H)), _full_spec((1, D_H)),
        _full_spec((D_H, D_OUT_PAD)),
    ],
    out_specs=[_row_spec(D_H), _row_spec(D_OUT_PAD)],
    out_shape=[
        jax.ShapeDtypeStruct((N, D_H), jnp.float32),
        jax.ShapeDtypeStruct((N, D_OUT_PAD), jnp.float32),
    ],
)

_tc_c = pl.pallas_call(
    _tc_c_body,
    grid=(GM,),
    in_specs=[
        _row_spec(D_OUT_PAD), _row_spec(D_OUT_PAD), _row_spec(D_OUT_PAD),
        _row_spec(1), _full_spec((1, D_OUT)),
    ],
    out_specs=_row_spec(D_OUT),
    out_shape=jax.ShapeDtypeStruct((N, D_OUT), jnp.float32),
)


@jax.jit
def kernel(x, edge_index, W1, b1, W2, b2):
    src = edge_index[0]
    dst = edge_index[1]
    deg_parts = _sc_deg(dst).reshape(NW, N)
    dinv, xs1 = _tc_a(deg_parts, x)
    agg1 = _sc_agg(xs1, src, dst)
    w2p = jnp.zeros((D_H, D_OUT_PAD), jnp.float32).at[:, :D_OUT].set(W2)
    h1, xs2 = _tc_b(agg1[0], agg1[1], xs1, dinv, W1,
                    b1.reshape(1, D_H), w2p)
    agg2 = _sc_agg2(xs2, src, dst)
    out = _tc_c(agg2[0], agg2[1], xs2, dinv, b2.reshape(1, D_OUT))
    return (out, h1)
